# baseline (speedup 1.0000x reference)
import functools

import jax
import jax.numpy as jnp
import numpy as np
from jax import lax
from jax.experimental import pallas as pl
from jax.experimental.pallas import tpu as pltpu

_P = 4
_C = 3
_BITS = 2
_MPV = 1.0
_MEAN = (0.5, 0.5, 0.5)
_STD = (0.5, 0.5, 0.5)


def _fused_kernel(tgt_ref, p_ref, q_ref, pred_ref, ma_ref, mb_ref,
                  h1a_ref, a1a_ref, h1b_ref, a1b_ref,
                  num_ref, den_ref, *, h, w, K, thr, edges):
    lab = None
    for ci in range(_C):
        img = jnp.minimum(tgt_ref[0, ci].astype(jnp.float32), thr)
        t1 = jnp.dot(img, p_ref[...], preferred_element_type=jnp.float32)
        s = jnp.dot(q_ref[...], t1, preferred_element_type=jnp.float32)
        d = (s > edges[0]).astype(jnp.float32)
        for e in edges[1:]:
            d = d + (s > e).astype(jnp.float32)
        lab = d if ci == 0 else lab + float((2 ** _BITS) ** ci) * d
    lmat = lab[:, 0:w]

    mida = jnp.dot(h1a_ref[...], lmat, preferred_element_type=jnp.float32)
    laba = jnp.sum(mida * a1a_ref[...], axis=1, keepdims=True)
    midb = jnp.dot(h1b_ref[...], lmat, preferred_element_type=jnp.float32)
    labb = jnp.sum(midb * a1b_ref[...], axis=1, keepdims=True)

    y = pred_ref[...]
    lane = lax.broadcasted_iota(jnp.int32, (1, 2 * K), 1)
    isa = lane < K
    kmod = jnp.where(isa, lane, lane - K)
    neg = jnp.float32(-1e30)
    mxa = jnp.max(jnp.where(isa, y, neg), axis=-1, keepdims=True)
    mxb = jnp.max(jnp.where(isa, neg, y), axis=-1, keepdims=True)
    sh = y - jnp.where(isa, mxa, mxb)
    e = jnp.exp(sh)
    suma = jnp.sum(jnp.where(isa, e, 0.0), axis=-1, keepdims=True)
    sumb = jnp.sum(e, axis=-1, keepdims=True) - suma
    labrow = jnp.where(isa, laba, labb).astype(jnp.int32)
    selhit = jnp.where(kmod == labrow, sh, 0.0)
    sela = jnp.sum(jnp.where(isa, selhit, 0.0), axis=-1, keepdims=True)
    selb = jnp.sum(selhit, axis=-1, keepdims=True) - sela
    cea = jnp.log(suma) - sela
    ceb = jnp.log(sumb) - selb

    ma = ma_ref[0]
    mb = mb_ref[0]
    num = (jnp.dot(ma, cea, preferred_element_type=jnp.float32)
           + jnp.dot(mb, ceb, preferred_element_type=jnp.float32))
    den = jnp.sum(ma) + jnp.sum(mb)
    num_ref[...] = jnp.broadcast_to(jnp.reshape(num, (1, 1, 1)), num_ref.shape)
    den_ref[...] = jnp.broadcast_to(jnp.reshape(den, (1, 1, 1)), den_ref.shape)


def kernel(predicted_patches, target, mask):
    b, c, H, W = target.shape
    p = _P
    h, w = H // p, W // p
    n = h * w
    K = predicted_patches.shape[-1]
    nh = n // 2

    thr = (_MPV - _MEAN[0]) / _STD[0]
    bin_size = _MPV / (2 ** _BITS)
    edges = tuple((float(e) - _MEAN[0]) / _STD[0]
                  for e in np.arange(bin_size, _MPV, bin_size))

    x = np.arange(W)
    p_np = np.zeros((W, 128), np.float32)
    p_np[x, x // p] = 1.0 / p
    q_np = np.zeros((h, H), np.float32)
    q_np[x[:H] // p, x[:H]] = 1.0 / p
    p_mat = jnp.asarray(p_np)
    q_mat = jnp.asarray(q_np)

    ra = 2 * np.arange(nh)
    rb = ra + 1
    def flat_consts(r):
        h1 = (r[:, None] // w == np.arange(h)[None, :]).astype(np.float32)
        a1 = (r[:, None] % w == np.arange(w)[None, :]).astype(np.float32)
        return jnp.asarray(h1), jnp.asarray(a1)
    h1a, a1a = flat_consts(ra)
    h1b, a1b = flat_consts(rb)

    pred2 = predicted_patches.reshape(b * nh, 2 * K)
    m3 = mask.reshape(b, nh, 2)
    ma = m3[:, :, 0].reshape(b, 1, nh).astype(jnp.float32)
    mb = m3[:, :, 1].reshape(b, 1, nh).astype(jnp.float32)

    fused = functools.partial(_fused_kernel, h=h, w=w, K=K, thr=thr,
                              edges=edges)
    num_parts, den_parts = pl.pallas_call(
        fused,
        out_shape=(jax.ShapeDtypeStruct((b, 8, 128), jnp.float32),
                   jax.ShapeDtypeStruct((b, 8, 128), jnp.float32)),
        grid=(b,),
        in_specs=[pl.BlockSpec((1, c, H, W), lambda i: (i, 0, 0, 0)),
                  pl.BlockSpec((W, 128), lambda i: (0, 0)),
                  pl.BlockSpec((h, H), lambda i: (0, 0)),
                  pl.BlockSpec((nh, 2 * K), lambda i: (i, 0)),
                  pl.BlockSpec((1, 1, nh), lambda i: (i, 0, 0)),
                  pl.BlockSpec((1, 1, nh), lambda i: (i, 0, 0)),
                  pl.BlockSpec((nh, h), lambda i: (0, 0)),
                  pl.BlockSpec((nh, w), lambda i: (0, 0)),
                  pl.BlockSpec((nh, h), lambda i: (0, 0)),
                  pl.BlockSpec((nh, w), lambda i: (0, 0))],
        out_specs=(pl.BlockSpec((1, 8, 128), lambda i: (i, 0, 0)),
                   pl.BlockSpec((1, 8, 128), lambda i: (i, 0, 0))),
        compiler_params=pltpu.CompilerParams(
            dimension_semantics=("parallel",),
            vmem_limit_bytes=56 * 1024 * 1024),
    )(target, p_mat, q_mat, pred2, ma, mb, h1a, a1a, h1b, a1b)

    return num_parts[:, 0, 0].sum() / den_parts[:, 0, 0].sum()


# speedup vs baseline: 1.0015x; 1.0015x over previous
import functools

import jax
import jax.numpy as jnp
import numpy as np
from jax import lax
from jax.experimental import pallas as pl
from jax.experimental.pallas import tpu as pltpu

_P = 4
_C = 3
_BITS = 2
_MPV = 1.0
_MEAN = (0.5, 0.5, 0.5)
_STD = (0.5, 0.5, 0.5)


def _fused_kernel(tgt_ref, p_ref, q_ref, pred_ref, ma_ref, mb_ref,
                  h1a_ref, a1a_ref, h1b_ref, a1b_ref,
                  num_ref, den_ref, *, h, w, K, thr, edges):
    lab = None
    for ci in range(_C):
        img = jnp.minimum(tgt_ref[0, ci].astype(jnp.float32), thr)
        t1 = jnp.dot(img, p_ref[...], preferred_element_type=jnp.float32)
        s = jnp.dot(q_ref[...], t1, preferred_element_type=jnp.float32)
        d = (s > edges[0]).astype(jnp.float32)
        for e in edges[1:]:
            d = d + (s > e).astype(jnp.float32)
        lab = d if ci == 0 else lab + float((2 ** _BITS) ** ci) * d
    lmat = lab[:, 0:w]

    mida = jnp.dot(h1a_ref[...], lmat, preferred_element_type=jnp.float32)
    laba = jnp.sum(mida * a1a_ref[...], axis=1, keepdims=True)
    midb = jnp.dot(h1b_ref[...], lmat, preferred_element_type=jnp.float32)
    labb = jnp.sum(midb * a1b_ref[...], axis=1, keepdims=True)

    y = pred_ref[0]
    lane = lax.broadcasted_iota(jnp.int32, (1, 2 * K), 1)
    isa = lane < K
    kmod = jnp.where(isa, lane, lane - K)
    neg = jnp.float32(-1e30)
    mxa = jnp.max(jnp.where(isa, y, neg), axis=-1, keepdims=True)
    mxb = jnp.max(jnp.where(isa, neg, y), axis=-1, keepdims=True)
    sh = y - jnp.where(isa, mxa, mxb)
    e = jnp.exp(sh)
    suma = jnp.sum(jnp.where(isa, e, 0.0), axis=-1, keepdims=True)
    sumb = jnp.sum(e, axis=-1, keepdims=True) - suma
    labrow = jnp.where(isa, laba, labb).astype(jnp.int32)
    selhit = jnp.where(kmod == labrow, sh, 0.0)
    sela = jnp.sum(jnp.where(isa, selhit, 0.0), axis=-1, keepdims=True)
    selb = jnp.sum(selhit, axis=-1, keepdims=True) - sela
    cea = jnp.log(suma) - sela
    ceb = jnp.log(sumb) - selb

    ma = ma_ref[0]
    mb = mb_ref[0]
    num = (jnp.dot(ma, cea, preferred_element_type=jnp.float32)
           + jnp.dot(mb, ceb, preferred_element_type=jnp.float32))
    den = jnp.sum(ma) + jnp.sum(mb)
    num_ref[...] = jnp.broadcast_to(jnp.reshape(num, (1, 1, 1)), num_ref.shape)
    den_ref[...] = jnp.broadcast_to(jnp.reshape(den, (1, 1, 1)), den_ref.shape)


def kernel(predicted_patches, target, mask):
    b, c, H, W = target.shape
    p = _P
    h, w = H // p, W // p
    n = h * w
    K = predicted_patches.shape[-1]
    nh = n // 2

    thr = (_MPV - _MEAN[0]) / _STD[0]
    bin_size = _MPV / (2 ** _BITS)
    edges = tuple((float(e) - _MEAN[0]) / _STD[0]
                  for e in np.arange(bin_size, _MPV, bin_size))

    x = np.arange(W)
    p_np = np.zeros((W, 128), np.float32)
    p_np[x, x // p] = 1.0 / p
    q_np = np.zeros((h, H), np.float32)
    q_np[x[:H] // p, x[:H]] = 1.0 / p
    p_mat = jnp.asarray(p_np)
    q_mat = jnp.asarray(q_np)

    ra = 2 * np.arange(nh)
    rb = ra + 1
    def flat_consts(r):
        h1 = (r[:, None] // w == np.arange(h)[None, :]).astype(np.float32)
        a1 = (r[:, None] % w == np.arange(w)[None, :]).astype(np.float32)
        return jnp.asarray(h1), jnp.asarray(a1)
    h1a, a1a = flat_consts(ra)
    h1b, a1b = flat_consts(rb)

    pred2 = predicted_patches.reshape(b, nh, 2 * K)
    m3 = mask.reshape(b, nh, 2)
    ma = m3[:, :, 0].reshape(b, 1, nh).astype(jnp.float32)
    mb = m3[:, :, 1].reshape(b, 1, nh).astype(jnp.float32)

    fused = functools.partial(_fused_kernel, h=h, w=w, K=K, thr=thr,
                              edges=edges)
    num_parts, den_parts = pl.pallas_call(
        fused,
        out_shape=(jax.ShapeDtypeStruct((b, 8, 128), jnp.float32),
                   jax.ShapeDtypeStruct((b, 8, 128), jnp.float32)),
        grid=(b,),
        in_specs=[pl.BlockSpec((1, c, H, W), lambda i: (i, 0, 0, 0)),
                  pl.BlockSpec((W, 128), lambda i: (0, 0)),
                  pl.BlockSpec((h, H), lambda i: (0, 0)),
                  pl.BlockSpec((1, nh, 2 * K), lambda i: (i, 0, 0)),
                  pl.BlockSpec((1, 1, nh), lambda i: (i, 0, 0)),
                  pl.BlockSpec((1, 1, nh), lambda i: (i, 0, 0)),
                  pl.BlockSpec((nh, h), lambda i: (0, 0)),
                  pl.BlockSpec((nh, w), lambda i: (0, 0)),
                  pl.BlockSpec((nh, h), lambda i: (0, 0)),
                  pl.BlockSpec((nh, w), lambda i: (0, 0))],
        out_specs=(pl.BlockSpec((1, 8, 128), lambda i: (i, 0, 0)),
                   pl.BlockSpec((1, 8, 128), lambda i: (i, 0, 0))),
        compiler_params=pltpu.CompilerParams(
            dimension_semantics=("parallel",),
            vmem_limit_bytes=56 * 1024 * 1024),
    )(target, p_mat, q_mat, pred2, ma, mb, h1a, a1a, h1b, a1b)

    return num_parts[:, 0, 0].sum() / den_parts[:, 0, 0].sum()


# speedup vs baseline: 1.3132x; 1.3112x over previous
import functools

import jax
import jax.numpy as jnp
import numpy as np
from jax import lax
from jax.experimental import pallas as pl
from jax.experimental.pallas import tpu as pltpu

_P = 4
_C = 3
_BITS = 2
_MPV = 1.0
_MEAN = (0.5, 0.5, 0.5)
_STD = (0.5, 0.5, 0.5)


def _fused_kernel(tgt_ref, p_ref, q_ref, pred_ref, m_ref, h1_ref, a1_ref,
                  num_ref, den_ref, *, h, w, thr, edges):
    lab = None
    for ci in range(_C):
        img = jnp.minimum(tgt_ref[0, ci].astype(jnp.float32), thr)
        t1 = jnp.dot(img, p_ref[...], preferred_element_type=jnp.float32)
        s = jnp.dot(q_ref[...], t1, preferred_element_type=jnp.float32)
        d = (s > edges[0]).astype(jnp.float32)
        for e in edges[1:]:
            d = d + (s > e).astype(jnp.float32)
        lab = d if ci == 0 else lab + float((2 ** _BITS) ** ci) * d
    lmat = lab[:, 0:w]

    mid = jnp.dot(h1_ref[...], lmat, preferred_element_type=jnp.float32)
    labcol = jnp.sum(mid * a1_ref[...], axis=1, keepdims=True)

    logits = pred_ref[0]
    mx = jnp.max(logits, axis=-1, keepdims=True)
    sh = logits - mx
    cls = lax.broadcasted_iota(jnp.int32, (1, logits.shape[-1]), 1)
    labi = labcol.astype(jnp.int32)
    sel = jnp.sum(jnp.where(cls == labi, sh, 0.0), axis=-1, keepdims=True)
    lse = jnp.log(jnp.sum(jnp.exp(sh), axis=-1, keepdims=True))
    ce = lse - sel

    m = m_ref[0]
    num = jnp.dot(m, ce, preferred_element_type=jnp.float32)
    den = jnp.sum(m)
    num_ref[...] = jnp.broadcast_to(jnp.reshape(num, (1, 1, 1)), num_ref.shape)
    den_ref[...] = jnp.broadcast_to(jnp.reshape(den, (1, 1, 1)), den_ref.shape)


def kernel(predicted_patches, target, mask):
    b, c, H, W = target.shape
    p = _P
    h, w = H // p, W // p
    n = h * w
    K = predicted_patches.shape[-1]

    thr = (_MPV - _MEAN[0]) / _STD[0]
    bin_size = _MPV / (2 ** _BITS)
    edges = tuple((float(e) - _MEAN[0]) / _STD[0]
                  for e in np.arange(bin_size, _MPV, bin_size))

    x = np.arange(W)
    p_np = np.zeros((W, 128), np.float32)
    p_np[x, x // p] = 1.0 / p
    q_np = np.zeros((h, H), np.float32)
    q_np[x[:H] // p, x[:H]] = 1.0 / p
    p_mat = jnp.asarray(p_np)
    q_mat = jnp.asarray(q_np)

    r = np.arange(n)
    h1 = jnp.asarray((r[:, None] // w == np.arange(h)[None, :])
                     .astype(np.float32))
    a1 = jnp.asarray((r[:, None] % w == np.arange(w)[None, :])
                     .astype(np.float32))

    mlane = mask.reshape(b, 1, n).astype(jnp.float32)

    fused = functools.partial(_fused_kernel, h=h, w=w, thr=thr, edges=edges)
    num_parts, den_parts = pl.pallas_call(
        fused,
        out_shape=(jax.ShapeDtypeStruct((b, 8, 128), jnp.float32),
                   jax.ShapeDtypeStruct((b, 8, 128), jnp.float32)),
        grid=(b,),
        in_specs=[pl.BlockSpec((1, c, H, W), lambda i: (i, 0, 0, 0)),
                  pl.BlockSpec((W, 128), lambda i: (0, 0)),
                  pl.BlockSpec((h, H), lambda i: (0, 0)),
                  pl.BlockSpec((1, n, K), lambda i: (i, 0, 0)),
                  pl.BlockSpec((1, 1, n), lambda i: (i, 0, 0)),
                  pl.BlockSpec((n, h), lambda i: (0, 0)),
                  pl.BlockSpec((n, w), lambda i: (0, 0))],
        out_specs=(pl.BlockSpec((1, 8, 128), lambda i: (i, 0, 0)),
                   pl.BlockSpec((1, 8, 128), lambda i: (i, 0, 0))),
        compiler_params=pltpu.CompilerParams(
            dimension_semantics=("parallel",),
            vmem_limit_bytes=56 * 1024 * 1024),
    )(target, p_mat, q_mat, predicted_patches, mlane, h1, a1)

    return num_parts[:, 0, 0].sum() / den_parts[:, 0, 0].sum()
